# Initial kernel scaffold; baseline (speedup 1.0000x reference)
import jax
import jax.numpy as jnp
from jax.experimental import pallas as pl


def kernel(features, rois):
    raise NotImplementedError("write your pallas kernel here")


# speedup vs baseline: 4.9767x; 4.9767x over previous
import functools

import jax
import jax.numpy as jnp
from jax import lax
from jax.experimental import pallas as pl
from jax.experimental.pallas import tpu as pltpu
from jax.experimental.pallas import tpu_sc as plsc

_ALIGNED_H = 7
_ALIGNED_W = 7
_SPATIAL_SCALE = 0.0625
_NC = 2
_NS = 16


def _build_sc_call(B, C, H, W, n_pad, rois_per_w):
    AH, AW = _ALIGNED_H, _ALIGNED_W
    nbins = AH * AW
    mesh = plsc.VectorSubcoreMesh(core_axis_name="c", subcore_axis_name="s")

    @functools.partial(
        pl.kernel,
        mesh=mesh,
        compiler_params=pltpu.CompilerParams(needs_layout_passes=False),
        out_type=jax.ShapeDtypeStruct((n_pad, C * nbins), jnp.float32),
        scratch_types=[
            pltpu.VMEM((rois_per_w, 16), jnp.float32),
            pltpu.VMEM((16,), jnp.int32),
            pltpu.VMEM((16, C), jnp.float32),
            pltpu.VMEM((C * nbins,), jnp.float32),
            pltpu.SemaphoreType.DMA,
        ],
    )
    def sc_kernel(table_hbm, rois_hbm, out_hbm, rois_v, idx_v, buf_v,
                  outt_v, sem):
        wid = lax.axis_index("s") * _NC + lax.axis_index("c")
        base = wid * rois_per_w
        pltpu.sync_copy(rois_hbm.at[pl.ds(base, rois_per_w)], rois_v)

        lane = lax.iota(jnp.int32, 16)
        iy = (lane >> 3) & 1
        ix = (lane >> 2) & 1
        cy = (lane >> 1) & 1
        cx = lane & 1
        iy_f = iy.astype(jnp.float32)
        ix_f = ix.astype(jnp.float32)

        def roi_body(j, roi_carry):
            roi = rois_v[j]
            b_off = roi[0].astype(jnp.int32) * (H * W)
            x1 = roi[1] * _SPATIAL_SCALE
            y1 = roi[2] * _SPATIAL_SCALE
            x2 = roi[3] * _SPATIAL_SCALE
            y2 = roi[4] * _SPATIAL_SCALE
            bin_w = jnp.maximum(x2 - x1, 1.0) * (1.0 / AW)
            bin_h = jnp.maximum(y2 - y1, 1.0) * (1.0 / AH)

            def bin_body(i, carry):
                ph, pw = carry
                ys = y1 + (ph.astype(jnp.float32) + 0.25 + 0.5 * iy_f) * bin_h
                xs = x1 + (pw.astype(jnp.float32) + 0.25 + 0.5 * ix_f) * bin_w
                yc = jnp.clip(ys, 0.0, float(H - 1))
                xc = jnp.clip(xs, 0.0, float(W - 1))
                y0 = yc.astype(jnp.int32)
                x0 = xc.astype(jnp.int32)
                ly = yc - y0.astype(jnp.float32)
                lx = xc - x0.astype(jnp.float32)
                wy = jnp.where(cy == 1, ly, 1.0 - ly)
                wx = jnp.where(cx == 1, lx, 1.0 - lx)
                yi = jnp.minimum(y0 + cy, H - 1)
                xi = jnp.minimum(x0 + cx, W - 1)
                idx_v[...] = b_off + yi * W + xi
                w = wy * wx * 0.25
                pltpu.async_copy(table_hbm.at[idx_v], buf_v, sem).wait()
                ws = [w[r] for r in range(16)]
                col0 = lane * nbins + i
                for k in range(C // 16):
                    sl = pl.ds(k * 16, 16)
                    acc = ws[0] * buf_v[0, sl]
                    for r in range(1, 16):
                        acc = acc + ws[r] * buf_v[r, sl]
                    plsc.store_scatter(outt_v, [k * 16 * nbins + col0], acc)
                pw1 = pw + 1
                wrap = pw1 >= AW
                return (jnp.where(wrap, ph + 1, ph),
                        jnp.where(wrap, 0, pw1))

            lax.fori_loop(0, nbins, bin_body,
                          (jnp.int32(0), jnp.int32(0)))
            pltpu.sync_copy(outt_v, out_hbm.at[base + j])
            return roi_carry

        lax.fori_loop(0, rois_per_w, roi_body, 0)

    return sc_kernel


def kernel(features, rois):
    B, C, H, W = features.shape
    N = rois.shape[0]
    nw = _NC * _NS
    rois_per_w = -(-N // nw)
    n_pad = rois_per_w * nw
    table = jnp.transpose(features, (0, 2, 3, 1)).reshape(B * H * W, C)
    rois_p = jnp.zeros((n_pad, 16), jnp.float32).at[:N, :5].set(rois)
    out = _build_sc_call(B, C, H, W, n_pad, rois_per_w)(table, rois_p)
    return out[:N].reshape(N, C, _ALIGNED_H, _ALIGNED_W)


# speedup vs baseline: 8.5080x; 1.7095x over previous
import functools

import jax
import jax.numpy as jnp
from jax import lax
from jax.experimental import pallas as pl
from jax.experimental.pallas import tpu as pltpu
from jax.experimental.pallas import tpu_sc as plsc

_ALIGNED_H = 7
_ALIGNED_W = 7
_SPATIAL_SCALE = 0.0625
_NC = 2
_NS = 16


def _build_sc_call(B, C, H, W, n_pad, rois_per_w):
    AH, AW = _ALIGNED_H, _ALIGNED_W
    nbins = AH * AW
    mesh = plsc.VectorSubcoreMesh(core_axis_name="c", subcore_axis_name="s")

    @functools.partial(
        pl.kernel,
        mesh=mesh,
        compiler_params=pltpu.CompilerParams(needs_layout_passes=False),
        out_type=jax.ShapeDtypeStruct((n_pad, C * nbins), jnp.float32),
        scratch_types=[
            pltpu.VMEM((rois_per_w, 16), jnp.float32),
            pltpu.VMEM((16,), jnp.int32),
            pltpu.VMEM((16,), jnp.int32),
            pltpu.VMEM((16, C), jnp.float32),
            pltpu.VMEM((16, C), jnp.float32),
            pltpu.VMEM((C * nbins,), jnp.float32),
            pltpu.SemaphoreType.DMA,
            pltpu.SemaphoreType.DMA,
        ],
    )
    def sc_kernel(table_hbm, rois_hbm, out_hbm, rois_v, idx0_v, idx1_v,
                  buf0_v, buf1_v, outt_v, sem0, sem1):
        wid = lax.axis_index("s") * _NC + lax.axis_index("c")
        base = wid * rois_per_w
        pltpu.sync_copy(rois_hbm.at[pl.ds(base, rois_per_w)], rois_v)
        idxs = (idx0_v, idx1_v)
        bufs = (buf0_v, buf1_v)
        sems = (sem0, sem1)

        lane = lax.iota(jnp.int32, 16)
        iy = (lane >> 3) & 1
        ix = (lane >> 2) & 1
        cy = (lane >> 1) & 1
        cx = lane & 1
        iy_f = iy.astype(jnp.float32)
        ix_f = ix.astype(jnp.float32)

        def roi_body(j, roi_carry):
            roi = rois_v[j]
            b_off = roi[0].astype(jnp.int32) * (H * W)
            x1 = roi[1] * _SPATIAL_SCALE
            y1 = roi[2] * _SPATIAL_SCALE
            x2 = roi[3] * _SPATIAL_SCALE
            y2 = roi[4] * _SPATIAL_SCALE
            bin_w = jnp.maximum(x2 - x1, 1.0) * (1.0 / AW)
            bin_h = jnp.maximum(y2 - y1, 1.0) * (1.0 / AH)

            def coords(i):
                iv = jnp.full((16,), i, jnp.int32)
                phv = (iv.astype(jnp.float32) * (1.0 / AW)).astype(jnp.int32)
                pwv = iv - AW * phv
                ys = y1 + (phv.astype(jnp.float32) + 0.25 + 0.5 * iy_f) * bin_h
                xs = x1 + (pwv.astype(jnp.float32) + 0.25 + 0.5 * ix_f) * bin_w
                yc = jnp.clip(ys, 0.0, float(H - 1))
                xc = jnp.clip(xs, 0.0, float(W - 1))
                y0 = yc.astype(jnp.int32)
                x0 = xc.astype(jnp.int32)
                ly = yc - y0.astype(jnp.float32)
                lx = xc - x0.astype(jnp.float32)
                wy = jnp.where(cy == 1, ly, 1.0 - ly)
                wx = jnp.where(cx == 1, lx, 1.0 - lx)
                yi = jnp.minimum(y0 + cy, H - 1)
                xi = jnp.minimum(x0 + cx, W - 1)
                return b_off + yi * W + xi, wy * wx * 0.25

            def fire(i, slot):
                idx, _ = coords(i)
                idxs[slot][...] = idx
                return pltpu.async_copy(table_hbm.at[idxs[slot]],
                                        bufs[slot], sems[slot])

            def drain(slot):
                pltpu.make_async_copy(table_hbm.at[idxs[slot]],
                                      bufs[slot], sems[slot]).wait()

            def accumulate(i, slot):
                _, w = coords(i)
                ws = [w[r] for r in range(16)]
                col0 = lane * nbins + i
                for k in range(C // 16):
                    sl = pl.ds(k * 16, 16)
                    acc = ws[0] * bufs[slot][0, sl]
                    for r in range(1, 16):
                        acc = acc + ws[r] * bufs[slot][r, sl]
                    plsc.store_scatter(outt_v, [k * 16 * nbins + col0], acc)

            fire(jnp.int32(0), 0)

            def bin_pair(p, carry):
                b0 = 2 * p
                fire(b0 + 1, 1)
                drain(0)
                accumulate(b0, 0)
                fire(b0 + 2, 0)
                drain(1)
                accumulate(b0 + 1, 1)
                return carry

            lax.fori_loop(0, (nbins - 1) // 2, bin_pair, 0)
            drain(0)
            accumulate(jnp.int32(nbins - 1), 0)
            pltpu.sync_copy(outt_v, out_hbm.at[base + j])
            return roi_carry

        lax.fori_loop(0, rois_per_w, roi_body, 0)

    return sc_kernel


def kernel(features, rois):
    B, C, H, W = features.shape
    N = rois.shape[0]
    nw = _NC * _NS
    rois_per_w = -(-N // nw)
    n_pad = rois_per_w * nw
    table = jnp.transpose(features, (0, 2, 3, 1)).reshape(B * H * W, C)
    rois_p = jnp.zeros((n_pad, 16), jnp.float32).at[:N, :5].set(rois)
    out = _build_sc_call(B, C, H, W, n_pad, rois_per_w)(table, rois_p)
    return out[:N].reshape(N, C, _ALIGNED_H, _ALIGNED_W)


# speedup vs baseline: 10.8531x; 1.2756x over previous
import functools

import jax
import jax.numpy as jnp
from jax import lax
from jax.experimental import pallas as pl
from jax.experimental.pallas import tpu as pltpu
from jax.experimental.pallas import tpu_sc as plsc

_ALIGNED_H = 7
_ALIGNED_W = 7
_SPATIAL_SCALE = 0.0625
_NC = 2
_NS = 16


def _build_sc_call(B, C, H, W, n_pad, rois_per_w):
    AH, AW = _ALIGNED_H, _ALIGNED_W
    nbins = AH * AW
    grp = AW * 16
    mesh = plsc.VectorSubcoreMesh(core_axis_name="c", subcore_axis_name="s")

    @functools.partial(
        pl.kernel,
        mesh=mesh,
        compiler_params=pltpu.CompilerParams(needs_layout_passes=False),
        out_type=jax.ShapeDtypeStruct((n_pad, C * nbins), jnp.float32),
        scratch_types=[
            pltpu.VMEM((rois_per_w, 16), jnp.float32),
            pltpu.VMEM((grp,), jnp.int32),
            pltpu.VMEM((grp,), jnp.int32),
            pltpu.VMEM((grp, C), jnp.float32),
            pltpu.VMEM((grp, C), jnp.float32),
            pltpu.VMEM((C * nbins,), jnp.float32),
            pltpu.SemaphoreType.DMA,
            pltpu.SemaphoreType.DMA,
        ],
    )
    def sc_kernel(table_hbm, rois_hbm, out_hbm, rois_v, idx0_v, idx1_v,
                  buf0_v, buf1_v, outt_v, sem0, sem1):
        wid = lax.axis_index("s") * _NC + lax.axis_index("c")
        base = wid * rois_per_w
        pltpu.sync_copy(rois_hbm.at[pl.ds(base, rois_per_w)], rois_v)
        idxs = (idx0_v, idx1_v)
        bufs = (buf0_v, buf1_v)
        sems = (sem0, sem1)

        lane = lax.iota(jnp.int32, 16)
        iy = (lane >> 3) & 1
        ix = (lane >> 2) & 1
        cy = (lane >> 1) & 1
        cx = lane & 1
        iy_f = iy.astype(jnp.float32)
        ix_f = ix.astype(jnp.float32)

        def roi_body(j, roi_carry):
            roi = rois_v[j]
            b_off = roi[0].astype(jnp.int32) * (H * W)
            x1 = roi[1] * _SPATIAL_SCALE
            y1 = roi[2] * _SPATIAL_SCALE
            x2 = roi[3] * _SPATIAL_SCALE
            y2 = roi[4] * _SPATIAL_SCALE
            bin_w = jnp.maximum(x2 - x1, 1.0) * (1.0 / AW)
            bin_h = jnp.maximum(y2 - y1, 1.0) * (1.0 / AH)

            def coords(ph, pw):
                phf = jnp.full((16,), ph, jnp.int32).astype(jnp.float32)
                pwf = jnp.full((16,), pw, jnp.int32).astype(jnp.float32)
                ys = y1 + (phf + 0.25 + 0.5 * iy_f) * bin_h
                xs = x1 + (pwf + 0.25 + 0.5 * ix_f) * bin_w
                yc = jnp.clip(ys, 0.0, float(H - 1))
                xc = jnp.clip(xs, 0.0, float(W - 1))
                y0 = yc.astype(jnp.int32)
                x0 = xc.astype(jnp.int32)
                ly = yc - y0.astype(jnp.float32)
                lx = xc - x0.astype(jnp.float32)
                wy = jnp.where(cy == 1, ly, 1.0 - ly)
                wx = jnp.where(cx == 1, lx, 1.0 - lx)
                yi = jnp.minimum(y0 + cy, H - 1)
                xi = jnp.minimum(x0 + cx, W - 1)
                return b_off + yi * W + xi, wy * wx * 0.25

            def fire(ph, slot):
                for q in range(AW):
                    idx, _ = coords(ph, q)
                    idxs[slot][pl.ds(q * 16, 16)] = idx
                pltpu.async_copy(table_hbm.at[idxs[slot]],
                                 bufs[slot], sems[slot])

            def drain(slot):
                pltpu.make_async_copy(table_hbm.at[idxs[slot]],
                                      bufs[slot], sems[slot]).wait()

            def acc_row(ph, slot):
                def acc_bin(q, carry):
                    _, w = coords(ph, q)
                    ws = [w[r] for r in range(16)]
                    col0 = lane * nbins + (AW * ph + q)
                    r0 = q * 16
                    for k in range(C // 16):
                        sl = pl.ds(k * 16, 16)
                        acc = ws[0] * bufs[slot][r0, sl]
                        for r in range(1, 16):
                            acc = acc + ws[r] * bufs[slot][r0 + r, sl]
                        plsc.store_scatter(outt_v, [k * 16 * nbins + col0],
                                           acc)
                    return carry

                lax.fori_loop(0, AW, acc_bin, 0)

            fire(jnp.int32(0), 0)

            def row_pair(p, carry):
                h0 = 2 * p
                fire(h0 + 1, 1)
                drain(0)
                acc_row(h0, 0)
                fire(h0 + 2, 0)
                drain(1)
                acc_row(h0 + 1, 1)
                return carry

            lax.fori_loop(0, (AH - 1) // 2, row_pair, 0)
            drain(0)
            acc_row(jnp.int32(AH - 1), 0)
            pltpu.sync_copy(outt_v, out_hbm.at[base + j])
            return roi_carry

        lax.fori_loop(0, rois_per_w, roi_body, 0)

    return sc_kernel


def kernel(features, rois):
    B, C, H, W = features.shape
    N = rois.shape[0]
    nw = _NC * _NS
    rois_per_w = -(-N // nw)
    n_pad = rois_per_w * nw
    table = jnp.transpose(features, (0, 2, 3, 1)).reshape(B * H * W, C)
    rois_p = jnp.zeros((n_pad, 16), jnp.float32).at[:N, :5].set(rois)
    out = _build_sc_call(B, C, H, W, n_pad, rois_per_w)(table, rois_p)
    return out[:N].reshape(N, C, _ALIGNED_H, _ALIGNED_W)


# speedup vs baseline: 11.4618x; 1.0561x over previous
import functools

import jax
import jax.numpy as jnp
from jax import lax
from jax.experimental import pallas as pl
from jax.experimental.pallas import tpu as pltpu
from jax.experimental.pallas import tpu_sc as plsc

_ALIGNED_H = 7
_ALIGNED_W = 7
_SPATIAL_SCALE = 0.0625
_NC = 2
_NS = 16


def _build_sc_call(B, C, H, W, n_pad, rois_per_w):
    AH, AW = _ALIGNED_H, _ALIGNED_W
    nbins = AH * AW
    grp = AW * 16
    mesh = plsc.VectorSubcoreMesh(core_axis_name="c", subcore_axis_name="s")

    @functools.partial(
        pl.kernel,
        mesh=mesh,
        compiler_params=pltpu.CompilerParams(needs_layout_passes=False),
        out_type=jax.ShapeDtypeStruct((n_pad, C * nbins), jnp.float32),
        scratch_types=[
            pltpu.VMEM((rois_per_w, 16), jnp.float32),
            pltpu.VMEM((grp,), jnp.int32),
            pltpu.VMEM((grp,), jnp.int32),
            pltpu.VMEM((grp, C), jnp.float32),
            pltpu.VMEM((grp, C), jnp.float32),
            pltpu.VMEM((C * nbins,), jnp.float32),
            pltpu.SemaphoreType.DMA,
            pltpu.SemaphoreType.DMA,
        ],
    )
    def sc_kernel(table_hbm, rois_hbm, out_hbm, rois_v, idx0_v, idx1_v,
                  buf0_v, buf1_v, outt_v, sem0, sem1):
        wid = lax.axis_index("s") * _NC + lax.axis_index("c")
        base = wid * rois_per_w
        pltpu.sync_copy(rois_hbm.at[pl.ds(base, rois_per_w)], rois_v)
        idxs = (idx0_v, idx1_v)
        bufs = (buf0_v, buf1_v)
        sems = (sem0, sem1)

        lane = lax.iota(jnp.int32, 16)
        iy = (lane >> 3) & 1
        ix = (lane >> 2) & 1
        cy = (lane >> 1) & 1
        cx = lane & 1
        iy_f = iy.astype(jnp.float32)
        ix_f = ix.astype(jnp.float32)

        def roi_body(j, roi_carry):
            roi = rois_v[j]
            b_off = roi[0].astype(jnp.int32) * (H * W)
            x1 = roi[1] * _SPATIAL_SCALE
            y1 = roi[2] * _SPATIAL_SCALE
            x2 = roi[3] * _SPATIAL_SCALE
            y2 = roi[4] * _SPATIAL_SCALE
            bin_w = jnp.maximum(x2 - x1, 1.0) * (1.0 / AW)
            bin_h = jnp.maximum(y2 - y1, 1.0) * (1.0 / AH)

            def coords(ph, pw):
                phf = jnp.full((16,), ph, jnp.int32).astype(jnp.float32)
                pwf = jnp.full((16,), pw, jnp.int32).astype(jnp.float32)
                ys = y1 + (phf + 0.25 + 0.5 * iy_f) * bin_h
                xs = x1 + (pwf + 0.25 + 0.5 * ix_f) * bin_w
                yc = jnp.clip(ys, 0.0, float(H - 1))
                xc = jnp.clip(xs, 0.0, float(W - 1))
                y0 = yc.astype(jnp.int32)
                x0 = xc.astype(jnp.int32)
                ly = yc - y0.astype(jnp.float32)
                lx = xc - x0.astype(jnp.float32)
                wy = jnp.where(cy == 1, ly, 1.0 - ly)
                wx = jnp.where(cx == 1, lx, 1.0 - lx)
                yi = jnp.minimum(y0 + cy, H - 1)
                xi = jnp.minimum(x0 + cx, W - 1)
                return b_off + yi * W + xi, wy * wx * 0.25

            def fire(ph, slot):
                for q in range(AW):
                    idx, _ = coords(ph, q)
                    idxs[slot][pl.ds(q * 16, 16)] = idx
                pltpu.async_copy(table_hbm.at[idxs[slot]],
                                 bufs[slot], sems[slot])

            def drain(slot):
                pltpu.make_async_copy(table_hbm.at[idxs[slot]],
                                      bufs[slot], sems[slot]).wait()

            def acc_row(ph, slot):
                def acc_bin(q, carry):
                    _, w = coords(ph, q)
                    ws = [w[r] for r in range(16)]
                    col0 = lane * nbins + (AW * ph + q)
                    r0 = q * 16
                    for k in range(C // 16):
                        sl = pl.ds(k * 16, 16)
                        terms = [ws[r] * bufs[slot][r0 + r, sl]
                                 for r in range(16)]
                        while len(terms) > 1:
                            terms = [a + b for a, b in
                                     zip(terms[::2], terms[1::2])]
                        plsc.store_scatter(outt_v, [k * 16 * nbins + col0],
                                           terms[0])
                    return carry

                lax.fori_loop(0, AW, acc_bin, 0)

            fire(jnp.int32(0), 0)

            def row_pair(p, carry):
                h0 = 2 * p
                fire(h0 + 1, 1)
                drain(0)
                acc_row(h0, 0)
                fire(h0 + 2, 0)
                drain(1)
                acc_row(h0 + 1, 1)
                return carry

            lax.fori_loop(0, (AH - 1) // 2, row_pair, 0)
            drain(0)
            acc_row(jnp.int32(AH - 1), 0)
            pltpu.sync_copy(outt_v, out_hbm.at[base + j])
            return roi_carry

        lax.fori_loop(0, rois_per_w, roi_body, 0)

    return sc_kernel


def kernel(features, rois):
    B, C, H, W = features.shape
    N = rois.shape[0]
    nw = _NC * _NS
    rois_per_w = -(-N // nw)
    n_pad = rois_per_w * nw
    table = jnp.transpose(features, (0, 2, 3, 1)).reshape(B * H * W, C)
    rois_p = jnp.zeros((n_pad, 16), jnp.float32).at[:N, :5].set(rois)
    out = _build_sc_call(B, C, H, W, n_pad, rois_per_w)(table, rois_p)
    return out[:N].reshape(N, C, _ALIGNED_H, _ALIGNED_W)


# speedup vs baseline: 12.7211x; 1.1099x over previous
import functools

import jax
import jax.numpy as jnp
from jax import lax
from jax.experimental import pallas as pl
from jax.experimental.pallas import tpu as pltpu
from jax.experimental.pallas import tpu_sc as plsc

_ALIGNED_H = 7
_ALIGNED_W = 7
_SPATIAL_SCALE = 0.0625
_NC = 2
_NS = 16


def _build_sc_call(B, C, H, W, N):
    AH, AW = _ALIGNED_H, _ALIGNED_W
    nbins = AH * AW
    grp = AW * 16
    nw = _NC * _NS
    stage = -(-N // nw)
    mesh = plsc.VectorSubcoreMesh(core_axis_name="c", subcore_axis_name="s")

    @functools.partial(
        pl.kernel,
        mesh=mesh,
        compiler_params=pltpu.CompilerParams(needs_layout_passes=False),
        out_type=jax.ShapeDtypeStruct((N, C * nbins), jnp.float32),
        scratch_types=[
            pltpu.VMEM((stage, 16), jnp.float32),
            pltpu.VMEM((grp,), jnp.int32),
            pltpu.VMEM((grp,), jnp.int32),
            pltpu.VMEM((grp, C), jnp.float32),
            pltpu.VMEM((grp, C), jnp.float32),
            pltpu.VMEM((C * nbins,), jnp.float32),
            pltpu.SemaphoreType.DMA,
            pltpu.SemaphoreType.DMA,
        ],
    )
    def sc_kernel(table_hbm, rois_hbm, out_hbm, rois_v, idx0_v, idx1_v,
                  buf0_v, buf1_v, outt_v, sem0, sem1):
        wid = lax.axis_index("s") * _NC + lax.axis_index("c")
        base = jnp.minimum(wid * stage, N - stage)
        pltpu.sync_copy(rois_hbm.at[pl.ds(base, stage)], rois_v)
        idxs = (idx0_v, idx1_v)
        bufs = (buf0_v, buf1_v)
        sems = (sem0, sem1)

        lane = lax.iota(jnp.int32, 16)
        iy = (lane >> 3) & 1
        ix = (lane >> 2) & 1
        cy = (lane >> 1) & 1
        cx = lane & 1
        iy_f = iy.astype(jnp.float32)
        ix_f = ix.astype(jnp.float32)

        def roi_body(j, roi_carry):
            roi = rois_v[j]
            b_off = roi[0].astype(jnp.int32) * (H * W)
            x1 = roi[1] * _SPATIAL_SCALE
            y1 = roi[2] * _SPATIAL_SCALE
            x2 = roi[3] * _SPATIAL_SCALE
            y2 = roi[4] * _SPATIAL_SCALE
            bin_w = jnp.maximum(x2 - x1, 1.0) * (1.0 / AW)
            bin_h = jnp.maximum(y2 - y1, 1.0) * (1.0 / AH)

            def coords(ph, pw):
                phf = jnp.full((16,), ph, jnp.int32).astype(jnp.float32)
                pwf = jnp.full((16,), pw, jnp.int32).astype(jnp.float32)
                ys = y1 + (phf + 0.25 + 0.5 * iy_f) * bin_h
                xs = x1 + (pwf + 0.25 + 0.5 * ix_f) * bin_w
                yc = jnp.clip(ys, 0.0, float(H - 1))
                xc = jnp.clip(xs, 0.0, float(W - 1))
                y0 = yc.astype(jnp.int32)
                x0 = xc.astype(jnp.int32)
                ly = yc - y0.astype(jnp.float32)
                lx = xc - x0.astype(jnp.float32)
                wy = jnp.where(cy == 1, ly, 1.0 - ly)
                wx = jnp.where(cx == 1, lx, 1.0 - lx)
                yi = jnp.minimum(y0 + cy, H - 1)
                xi = jnp.minimum(x0 + cx, W - 1)
                return b_off + yi * W + xi, wy * wx * 0.25

            def fire(ph, slot):
                for q in range(AW):
                    idx, _ = coords(ph, q)
                    idxs[slot][pl.ds(q * 16, 16)] = idx
                pltpu.async_copy(table_hbm.at[idxs[slot]],
                                 bufs[slot], sems[slot])

            def drain(slot):
                pltpu.make_async_copy(table_hbm.at[idxs[slot]],
                                      bufs[slot], sems[slot]).wait()

            def acc_row(ph, slot):
                def acc_bin(q, carry):
                    _, w = coords(ph, q)
                    ws = [w[r] for r in range(16)]
                    col0 = lane * nbins + (AW * ph + q)
                    r0 = q * 16
                    for k in range(C // 16):
                        sl = pl.ds(k * 16, 16)
                        terms = [ws[r] * bufs[slot][r0 + r, sl]
                                 for r in range(16)]
                        while len(terms) > 1:
                            terms = [a + b for a, b in
                                     zip(terms[::2], terms[1::2])]
                        plsc.store_scatter(outt_v, [k * 16 * nbins + col0],
                                           terms[0])
                    return carry

                lax.fori_loop(0, AW, acc_bin, 0)

            fire(jnp.int32(0), 0)

            def row_pair(p, carry):
                h0 = 2 * p
                fire(h0 + 1, 1)
                drain(0)
                acc_row(h0, 0)
                fire(h0 + 2, 0)
                drain(1)
                acc_row(h0 + 1, 1)
                return carry

            lax.fori_loop(0, (AH - 1) // 2, row_pair, 0)
            drain(0)
            acc_row(jnp.int32(AH - 1), 0)
            pltpu.sync_copy(outt_v, out_hbm.at[base + j])
            return roi_carry

        lax.fori_loop(0, stage, roi_body, 0)

    return sc_kernel


def kernel(features, rois):
    B, C, H, W = features.shape
    N = rois.shape[0]
    table = jnp.transpose(features, (0, 2, 3, 1)).reshape(B * H * W, C)
    rois_p = jnp.zeros((N, 16), jnp.float32).at[:, :5].set(rois)
    out = _build_sc_call(B, C, H, W, N)(table, rois_p)
    return out.reshape(N, C, _ALIGNED_H, _ALIGNED_W)
